# baseline (speedup 1.0000x reference)
import functools

import jax
import jax.numpy as jnp
from jax import lax
from jax.experimental import pallas as pl
from jax.experimental.pallas import tpu as pltpu
from jax.experimental.pallas import tpu_sc as plsc

VOCAB = 1000000
EMBED = 64
BATCH = 16384
NEG_K = 20

NC = 2
NS = 16
NW = NC * NS

B_W = BATCH // NW

_mesh = plsc.VectorSubcoreMesh(
    core_axis_name="c", subcore_axis_name="s", num_cores=NC, num_subcores=NS
)


@functools.partial(
    pl.kernel,
    out_type=(
        jax.ShapeDtypeStruct((BATCH, EMBED), jnp.float32),
        jax.ShapeDtypeStruct((BATCH, EMBED), jnp.float32),
        jax.ShapeDtypeStruct((BATCH, NEG_K * EMBED), jnp.float32),
    ),
    mesh=_mesh,
    scratch_types=[
        pltpu.VMEM((B_W,), jnp.int32),
        pltpu.VMEM((B_W,), jnp.int32),
        pltpu.VMEM((NEG_K, B_W), jnp.int32),
        pltpu.VMEM((B_W, EMBED), jnp.float32),
        pltpu.VMEM((B_W, EMBED), jnp.float32),
        pltpu.SemaphoreType.DMA,
        pltpu.SemaphoreType.DMA,
        pltpu.SemaphoreType.DMA,
    ],
    compiler_params=pltpu.CompilerParams(use_tc_tiling_on_sc=False),
)
def _sc_gather(center_hbm, pos_hbm, negt_hbm, in_hbm, out_hbm,
               o_center, o_pos, o_neg,
               idxc, idxp, idxn, bufa, bufb, semi, sema, semb):
    wid = lax.axis_index("s") * NC + lax.axis_index("c")
    base = pl.multiple_of(wid * B_W, B_W)

    di = pltpu.async_copy(center_hbm.at[pl.ds(base, B_W)], idxc, semi)
    dp = pltpu.async_copy(pos_hbm.at[pl.ds(base, B_W)], idxp, semi)
    dn = pltpu.async_copy(negt_hbm.at[:, pl.ds(base, B_W)], idxn, semi)
    di.wait()
    ga = pltpu.async_copy(in_hbm.at[idxc], bufa, sema)
    dp.wait()
    gb = pltpu.async_copy(out_hbm.at[idxp], bufb, semb)
    ga.wait()
    pltpu.sync_copy(bufa, o_center.at[pl.ds(base, B_W)])
    dn.wait()
    pltpu.async_copy(out_hbm.at[idxn.at[0]], bufa, sema)
    gb.wait()
    pltpu.sync_copy(bufb, o_pos.at[pl.ds(base, B_W)])
    pltpu.async_copy(out_hbm.at[idxn.at[1]], bufb, semb)

    def neg_out(k):
        return o_neg.at[pl.ds(base, B_W),
                        pl.ds(pl.multiple_of(k * EMBED, EMBED), EMBED)]

    def neg_pair(j, _):
        ka = j * 2
        pltpu.make_async_copy(out_hbm.at[pl.ds(0, B_W)], bufa, sema).wait()
        pltpu.sync_copy(bufa, neg_out(ka))
        pltpu.async_copy(out_hbm.at[idxn.at[ka + 2]], bufa, sema)
        pltpu.make_async_copy(out_hbm.at[pl.ds(0, B_W)], bufb, semb).wait()
        pltpu.sync_copy(bufb, neg_out(ka + 1))
        pltpu.async_copy(out_hbm.at[idxn.at[ka + 3]], bufb, semb)
        return ()
    lax.fori_loop(0, (NEG_K - 2) // 2, neg_pair, ())

    pltpu.make_async_copy(out_hbm.at[pl.ds(0, B_W)], bufa, sema).wait()
    pltpu.sync_copy(bufa, neg_out(NEG_K - 2))
    pltpu.make_async_copy(out_hbm.at[pl.ds(0, B_W)], bufb, semb).wait()
    pltpu.sync_copy(bufb, neg_out(NEG_K - 1))


@jax.jit
def kernel(center_word, pos_word, neg_word, in_embed, out_embed):
    emb, pos, neg = _sc_gather(center_word, pos_word, neg_word.T,
                               in_embed, out_embed)
    return emb, pos, neg.reshape(BATCH, NEG_K, EMBED)
